# baseline (speedup 1.0000x reference)
import functools

import jax
import jax.numpy as jnp
from jax import lax
from jax.experimental import pallas as pl
from jax.experimental.pallas import tpu as pltpu
from jax.experimental.pallas import tpu_sc as plsc

VOCAB = 256
D = 512
B = 4 * 8192

_info = plsc.get_sparse_core_info()
NC = _info.num_cores
NS = _info.num_subcores
NW = NC * NS
B_PER_W = B // NW
CH = 64
NCH = B_PER_W // CH


def _make_kernel():
  mesh = plsc.VectorSubcoreMesh(core_axis_name="c", subcore_axis_name="s")

  @functools.partial(
      pl.kernel,
      mesh=mesh,
      out_type=jax.ShapeDtypeStruct((B, D), jnp.float32),
      scratch_types=[
          pltpu.VMEM((B_PER_W,), jnp.int32),
          pltpu.VMEM((CH, D), jnp.float32),
          pltpu.VMEM((CH, D), jnp.float32),
          pltpu.VMEM((CH, D), jnp.float32),
          pltpu.SemaphoreType.DMA,
          pltpu.SemaphoreType.DMA,
          pltpu.SemaphoreType.DMA,
          pltpu.SemaphoreType.DMA,
          pltpu.SemaphoreType.DMA,
          pltpu.SemaphoreType.DMA,
      ],
  )
  def body(x_hbm, table_hbm, out_hbm, idx_v, buf0, buf1, buf2,
           gs0, gs1, gs2, ss0, ss1, ss2):
    wid = lax.axis_index("s") * NC + lax.axis_index("c")
    base = wid * B_PER_W
    pltpu.sync_copy(x_hbm.at[pl.ds(base, B_PER_W)], idx_v)

    bufs = (buf0, buf1, buf2)
    gsems = (gs0, gs1, gs2)
    ssems = (ss0, ss1, ss2)
    gh = [None, None, None]
    sh = [None, None, None]

    def gather(g):
      b = g % 3
      gh[b] = pltpu.async_copy(
          table_hbm.at[idx_v.at[pl.ds(g * CH, CH)]], bufs[b], gsems[b])

    gather(0)
    gather(1)
    for g in range(NCH):
      b = g % 3
      gh[b].wait()
      if g + 2 < NCH:
        gather(g + 2)
      pltpu.sync_copy(bufs[b], out_hbm.at[pl.ds(base + g * CH, CH)])

  return body


_kernel = _make_kernel()


def kernel(x, embedding):
  flat = jnp.reshape(x, (B,)).astype(jnp.int32)
  out = _kernel(flat, embedding)
  return jnp.reshape(out, (x.shape[0], x.shape[1], D))


# speedup vs baseline: 1.0035x; 1.0035x over previous
import functools

import jax
import jax.numpy as jnp
from jax import lax
from jax.experimental import pallas as pl
from jax.experimental.pallas import tpu as pltpu
from jax.experimental.pallas import tpu_sc as plsc

VOCAB = 256
D = 512
B = 4 * 8192

_info = plsc.get_sparse_core_info()
NC = _info.num_cores
NS = _info.num_subcores
NW = NC * NS
B_PER_W = B // NW
CH = 64
NCH = B_PER_W // CH


def _make_kernel():
  mesh = plsc.VectorSubcoreMesh(core_axis_name="c", subcore_axis_name="s")

  @functools.partial(
      pl.kernel,
      mesh=mesh,
      out_type=jax.ShapeDtypeStruct((B, D), jnp.float32),
      scratch_types=[
          pltpu.VMEM((B_PER_W,), jnp.int32),
          pltpu.VMEM((CH, D), jnp.float32),
          pltpu.VMEM((CH, D), jnp.float32),
          pltpu.VMEM((CH, D), jnp.float32),
          pltpu.SemaphoreType.DMA,
          pltpu.SemaphoreType.DMA,
          pltpu.SemaphoreType.DMA,
          pltpu.SemaphoreType.DMA,
          pltpu.SemaphoreType.DMA,
          pltpu.SemaphoreType.DMA,
      ],
  )
  def body(x_hbm, table_hbm, out_hbm, idx_v, buf0, buf1, buf2,
           gs0, gs1, gs2, ss0, ss1, ss2):
    wid = lax.axis_index("s") * NC + lax.axis_index("c")
    base = wid * B_PER_W
    pltpu.sync_copy(x_hbm.at[pl.ds(base, B_PER_W)], idx_v)

    bufs = (buf0, buf1, buf2)
    gsems = (gs0, gs1, gs2)
    ssems = (ss0, ss1, ss2)
    gh = [None, None, None]
    sh = [None, None, None]

    def gather(g):
      b = g % 3
      gh[b] = pltpu.async_copy(
          table_hbm.at[idx_v.at[pl.ds(g * CH, CH)]], bufs[b], gsems[b])

    gather(0)
    gather(1)
    for g in range(NCH):
      b = g % 3
      gh[b].wait()
      if g + 2 < NCH:
        if sh[(g + 2) % 3] is not None:
          sh[(g + 2) % 3].wait()
        gather(g + 2)
      sh[b] = pltpu.async_copy(
          bufs[b], out_hbm.at[pl.ds(base + g * CH, CH)], ssems[b])
    sh[(NCH - 2) % 3].wait()
    sh[(NCH - 1) % 3].wait()

  return body


_kernel = _make_kernel()


def kernel(x, embedding):
  flat = jnp.reshape(x, (B,)).astype(jnp.int32)
  out = _kernel(flat, embedding)
  return jnp.reshape(out, (x.shape[0], x.shape[1], D))
